# baseline (speedup 1.0000x reference)
import functools

import jax
import jax.numpy as jnp
from jax import lax
from jax.experimental import pallas as pl
from jax.experimental.pallas import tpu as pltpu
from jax.experimental.pallas import tpu_sc as plsc

N = 10000
E = 320000
D_NODE = 128
D_EDGE = 16
H = 4

NUM_CORES = 2
NUM_SUBCORES = 16
NUM_TILES = NUM_CORES * NUM_SUBCORES
EDGES_PER_TILE = E // NUM_TILES
HALF_N = N // 2

_sc_mesh = plsc.VectorSubcoreMesh(core_axis_name="c", subcore_axis_name="s")
_sc_params = pltpu.CompilerParams(use_tc_tiling_on_sc=False,
                                  needs_layout_passes=False)

_INV_SQRT2 = 0.7071067811865476


def _gelu_exact(x):
    return x * 0.5 * (1.0 + lax.erf(x * _INV_SQRT2))


_TP_ITERS = N // 16


@functools.partial(
    pl.kernel,
    out_type=(
        jax.ShapeDtypeStruct((H, N), jnp.float32),
        jax.ShapeDtypeStruct((H, N), jnp.float32),
    ),
    mesh=_sc_mesh,
    compiler_params=_sc_params,
    scratch_types=[
        pltpu.VMEM((EDGES_PER_TILE,), jnp.int32),
        pltpu.VMEM((EDGES_PER_TILE, H), jnp.float32),
        pltpu.VMEM((N,), jnp.float32),
        pltpu.VMEM_SHARED((N, H), jnp.float32),
    ],
)
def _sc_scatter(ei_hbm, attn_hbm, zeros_hbm, out0_hbm, out1_hbm,
                idx_v, vals_v, row_v, acc_sh):
    c = lax.axis_index("c")
    s = lax.axis_index("s")
    wid = c * NUM_SUBCORES + s

    @pl.when(s < 10)
    def _():
        sl = pl.ds(s * (N // 10), N // 10)
        pltpu.sync_copy(zeros_hbm.at[sl], acc_sh.at[sl])

    base = wid * EDGES_PER_TILE
    pltpu.sync_copy(ei_hbm.at[1].at[pl.ds(base, EDGES_PER_TILE)], idx_v)
    pltpu.sync_copy(attn_hbm.at[pl.ds(base, EDGES_PER_TILE)], vals_v)

    plsc.subcore_barrier()
    pltpu.sync_copy(vals_v, acc_sh.at[idx_v], add=True)
    plsc.subcore_barrier()

    @pl.when(s < H)
    def _():
        pltpu.sync_copy(acc_sh, vals_v)
        rows0 = lax.iota(jnp.int32, 16)
        cols = jnp.zeros((16,), jnp.int32) + s

        def body(i, carry):
            row_v[pl.ds(i * 16, 16)] = plsc.load_gather(
                vals_v, [rows0 + i * 16, cols])
            return carry

        lax.fori_loop(0, _TP_ITERS, body, 0)

        @pl.when(c == 0)
        def _():
            pltpu.sync_copy(row_v, out0_hbm.at[s])

        @pl.when(c == 1)
        def _():
            pltpu.sync_copy(row_v, out1_hbm.at[s])


def _node_mlp_body(p0_ref, p1_ref, nf_ref, wn1a_ref, wn1bt_ref, be1t_ref,
                   wn2_ref, bn2_ref, out_ref):
    napt = p0_ref[...] + p1_ref[...]
    hm = jnp.max(napt, axis=1, keepdims=True) + 1e-10
    napnt = napt / hm
    h1t = lax.dot_general(wn1a_ref[...], nf_ref[...],
                          (((0,), (1,)), ((), ())))
    for hh in range(H):
        h1t = h1t + wn1bt_ref[:, hh:hh + 1] * napnt[hh:hh + 1, :]
    h1t = h1t + be1t_ref[...]
    h1t = _gelu_exact(h1t)
    logits = jnp.sum(h1t * wn2_ref[...], axis=0, keepdims=True) + bn2_ref[...]
    out_ref[...] = jax.nn.sigmoid(logits)[0]


_node_mlp = pl.pallas_call(
    _node_mlp_body,
    out_shape=jax.ShapeDtypeStruct((N,), jnp.float32),
)


_GATHER_ITERS = EDGES_PER_TILE // 16


@functools.partial(
    pl.kernel,
    out_type=(
        jax.ShapeDtypeStruct((E,), jnp.float32),
        jax.ShapeDtypeStruct((E,), jnp.float32),
    ),
    mesh=_sc_mesh,
    compiler_params=_sc_params,
    scratch_types=[
        pltpu.VMEM_SHARED((N,), jnp.float32),
        pltpu.VMEM((EDGES_PER_TILE,), jnp.int32),
        pltpu.VMEM((EDGES_PER_TILE,), jnp.int32),
        pltpu.VMEM((EDGES_PER_TILE,), jnp.float32),
        pltpu.VMEM((EDGES_PER_TILE,), jnp.float32),
    ],
)
def _sc_gather(ei_hbm, gates_hbm, outs_hbm, outt_hbm,
               gates_sh, sidx_v, tidx_v, souts_v, soutt_v):
    c = lax.axis_index("c")
    s = lax.axis_index("s")
    wid = c * NUM_SUBCORES + s
    base = wid * EDGES_PER_TILE

    @pl.when(s < 10)
    def _():
        sl = pl.ds(s * (N // 10), N // 10)
        pltpu.sync_copy(gates_hbm.at[sl], gates_sh.at[sl])

    pltpu.sync_copy(ei_hbm.at[0].at[pl.ds(base, EDGES_PER_TILE)], sidx_v)
    pltpu.sync_copy(ei_hbm.at[1].at[pl.ds(base, EDGES_PER_TILE)], tidx_v)

    plsc.subcore_barrier()

    pltpu.sync_copy(gates_sh.at[sidx_v], souts_v)
    pltpu.sync_copy(gates_sh.at[tidx_v], soutt_v)

    pltpu.sync_copy(souts_v, outs_hbm.at[pl.ds(base, EDGES_PER_TILE)])
    pltpu.sync_copy(soutt_v, outt_hbm.at[pl.ds(base, EDGES_PER_TILE)])


EDGE_BLOCK = 64000
EDGE_GRID = E // EDGE_BLOCK


def _edge_mlp_body(eft_ref, attnt_ref, sg_ref, tg_ref, we1a_ref, we1b_ref,
                   we1c_ref, be1t_ref, we2_ref, be2_ref, out_ref):
    i = pl.program_id(0)
    esl = pl.ds(i * EDGE_BLOCK, EDGE_BLOCK)
    cdims = (((0,), (0,)), ((), ()))
    ht = lax.dot_general(we1a_ref[...], eft_ref[...], cdims)
    ht = ht + lax.dot_general(we1b_ref[...], attnt_ref[...], cdims)
    sgtg = jnp.concatenate(
        [sg_ref[esl].reshape(1, EDGE_BLOCK), tg_ref[esl].reshape(1, EDGE_BLOCK)],
        axis=0)
    ht = ht + lax.dot_general(we1c_ref[...], sgtg, cdims)
    ht = ht + be1t_ref[...]
    ht = _gelu_exact(ht)
    logits = lax.dot_general(we2_ref[...], ht, cdims) + be2_ref[...]
    out_ref[esl] = jax.nn.sigmoid(logits)[0]


_edge_mlp = pl.pallas_call(
    _edge_mlp_body,
    grid=(EDGE_GRID,),
    in_specs=[
        pl.BlockSpec((D_EDGE, EDGE_BLOCK), lambda i: (0, i)),
        pl.BlockSpec((H, EDGE_BLOCK), lambda i: (0, i)),
        pl.BlockSpec((E,), lambda i: (0,)),
        pl.BlockSpec((E,), lambda i: (0,)),
        pl.BlockSpec((D_EDGE, D_EDGE), lambda i: (0, 0)),
        pl.BlockSpec((H, D_EDGE), lambda i: (0, 0)),
        pl.BlockSpec((2, D_EDGE), lambda i: (0, 0)),
        pl.BlockSpec((D_EDGE, 1), lambda i: (0, 0)),
        pl.BlockSpec((D_EDGE, 1), lambda i: (0, 0)),
        pl.BlockSpec((1, 1), lambda i: (0, 0)),
    ],
    out_specs=pl.BlockSpec((E,), lambda i: (0,)),
    out_shape=jax.ShapeDtypeStruct((E,), jnp.float32),
    compiler_params=pltpu.CompilerParams(
        dimension_semantics=("parallel",)),
)


def kernel(node_features, edge_features, edge_index, node_attn_weights,
           edge_attn_weights, Wn1, bn1, Wn2, bn2, We1, be1, We2, be2):
    attn_t = node_attn_weights.T

    zeros = jnp.zeros((N, H), jnp.float32)
    p0, p1 = _sc_scatter(edge_index, node_attn_weights, zeros)

    node_gates = _node_mlp(
        p0, p1, node_features,
        Wn1[:D_NODE], Wn1[D_NODE:].T,
        bn1.reshape(-1, 1), Wn2, bn2.reshape(1, 1),
    )

    src_g, tgt_g = _sc_gather(edge_index, node_gates)

    edge_gates = _edge_mlp(
        edge_features.T, attn_t, src_g, tgt_g,
        We1[:D_EDGE], We1[D_EDGE:D_EDGE + H], We1[D_EDGE + H:],
        be1.reshape(-1, 1), We2, be2.reshape(1, 1),
    )

    return (node_gates, edge_gates)


# speedup vs baseline: 4.1748x; 4.1748x over previous
import functools

import jax
import jax.numpy as jnp
from jax import lax
from jax.experimental import pallas as pl
from jax.experimental.pallas import tpu as pltpu
from jax.experimental.pallas import tpu_sc as plsc

N = 10000
E = 320000
D_NODE = 128
D_EDGE = 16
H = 4

NUM_CORES = 2
NUM_SUBCORES = 16
NUM_TILES = NUM_CORES * NUM_SUBCORES
EDGES_PER_TILE = E // NUM_TILES
HALF_N = N // 2

_sc_mesh = plsc.VectorSubcoreMesh(core_axis_name="c", subcore_axis_name="s")
_sc_params = pltpu.CompilerParams(use_tc_tiling_on_sc=False,
                                  needs_layout_passes=False)

_INV_SQRT2 = 0.7071067811865476


def _gelu_exact(x):
    return x * 0.5 * (1.0 + lax.erf(x * _INV_SQRT2))


_TP_ITERS = N // 16


@functools.partial(
    pl.kernel,
    out_type=(
        jax.ShapeDtypeStruct((H, N), jnp.float32),
        jax.ShapeDtypeStruct((H, N), jnp.float32),
    ),
    mesh=_sc_mesh,
    compiler_params=_sc_params,
    scratch_types=[
        pltpu.VMEM((EDGES_PER_TILE,), jnp.int32),
        pltpu.VMEM((EDGES_PER_TILE,), jnp.float32),
        pltpu.VMEM_SHARED((H, N), jnp.float32),
    ],
)
def _sc_scatter(ei_hbm, attnt_hbm, zeros_hbm, out0_hbm, out1_hbm,
                idx_v, vals_v, acc_sh):
    c = lax.axis_index("c")
    s = lax.axis_index("s")
    wid = c * NUM_SUBCORES + s

    @pl.when(s < 2 * H)
    def _():
        h = s // 2
        sl = pl.ds((s % 2) * HALF_N, HALF_N)
        pltpu.sync_copy(zeros_hbm.at[h].at[sl], acc_sh.at[h].at[sl])

    base = wid * EDGES_PER_TILE
    pltpu.sync_copy(ei_hbm.at[1].at[pl.ds(base, EDGES_PER_TILE)], idx_v)

    plsc.subcore_barrier()

    for h in range(H):
        pltpu.sync_copy(attnt_hbm.at[h].at[pl.ds(base, EDGES_PER_TILE)],
                        vals_v)
        pltpu.sync_copy(vals_v, acc_sh.at[h].at[idx_v], add=True)

    plsc.subcore_barrier()

    @pl.when(s < 2 * H)
    def _():
        h = s // 2
        sl = pl.ds((s % 2) * HALF_N, HALF_N)

        @pl.when(c == 0)
        def _():
            pltpu.sync_copy(acc_sh.at[h].at[sl], out0_hbm.at[h].at[sl])

        @pl.when(c == 1)
        def _():
            pltpu.sync_copy(acc_sh.at[h].at[sl], out1_hbm.at[h].at[sl])


def _node_mlp_body(p0_ref, p1_ref, nf_ref, wn1a_ref, wn1bt_ref, be1t_ref,
                   wn2_ref, bn2_ref, out_ref):
    napt = p0_ref[...] + p1_ref[...]
    hm = jnp.max(napt, axis=1, keepdims=True) + 1e-10
    napnt = napt / hm
    h1t = lax.dot_general(wn1a_ref[...], nf_ref[...],
                          (((0,), (1,)), ((), ())))
    for hh in range(H):
        h1t = h1t + wn1bt_ref[:, hh:hh + 1] * napnt[hh:hh + 1, :]
    h1t = h1t + be1t_ref[...]
    h1t = _gelu_exact(h1t)
    logits = jnp.sum(h1t * wn2_ref[...], axis=0, keepdims=True) + bn2_ref[...]
    out_ref[...] = jax.nn.sigmoid(logits)[0]


_node_mlp = pl.pallas_call(
    _node_mlp_body,
    out_shape=jax.ShapeDtypeStruct((N,), jnp.float32),
)


_GATHER_ITERS = EDGES_PER_TILE // 16


@functools.partial(
    pl.kernel,
    out_type=(
        jax.ShapeDtypeStruct((E,), jnp.float32),
        jax.ShapeDtypeStruct((E,), jnp.float32),
    ),
    mesh=_sc_mesh,
    compiler_params=_sc_params,
    scratch_types=[
        pltpu.VMEM_SHARED((N,), jnp.float32),
        pltpu.VMEM((EDGES_PER_TILE,), jnp.int32),
        pltpu.VMEM((EDGES_PER_TILE,), jnp.int32),
        pltpu.VMEM((EDGES_PER_TILE,), jnp.float32),
        pltpu.VMEM((EDGES_PER_TILE,), jnp.float32),
    ],
)
def _sc_gather(ei_hbm, gates_hbm, outs_hbm, outt_hbm,
               gates_sh, sidx_v, tidx_v, souts_v, soutt_v):
    c = lax.axis_index("c")
    s = lax.axis_index("s")
    wid = c * NUM_SUBCORES + s
    base = wid * EDGES_PER_TILE

    @pl.when(s < 10)
    def _():
        sl = pl.ds(s * (N // 10), N // 10)
        pltpu.sync_copy(gates_hbm.at[sl], gates_sh.at[sl])

    pltpu.sync_copy(ei_hbm.at[0].at[pl.ds(base, EDGES_PER_TILE)], sidx_v)
    pltpu.sync_copy(ei_hbm.at[1].at[pl.ds(base, EDGES_PER_TILE)], tidx_v)

    plsc.subcore_barrier()

    pltpu.sync_copy(gates_sh.at[sidx_v], souts_v)
    pltpu.sync_copy(gates_sh.at[tidx_v], soutt_v)

    pltpu.sync_copy(souts_v, outs_hbm.at[pl.ds(base, EDGES_PER_TILE)])
    pltpu.sync_copy(soutt_v, outt_hbm.at[pl.ds(base, EDGES_PER_TILE)])


EDGE_BLOCK = 64000
EDGE_GRID = E // EDGE_BLOCK


def _edge_mlp_body(eft_ref, attnt_ref, sg_ref, tg_ref, we1a_ref, we1b_ref,
                   we1c_ref, be1t_ref, we2_ref, be2_ref, out_ref):
    i = pl.program_id(0)
    esl = pl.ds(i * EDGE_BLOCK, EDGE_BLOCK)
    cdims = (((0,), (0,)), ((), ()))
    ht = lax.dot_general(we1a_ref[...], eft_ref[...], cdims)
    ht = ht + lax.dot_general(we1b_ref[...], attnt_ref[...], cdims)
    sgtg = jnp.concatenate(
        [sg_ref[esl].reshape(1, EDGE_BLOCK), tg_ref[esl].reshape(1, EDGE_BLOCK)],
        axis=0)
    ht = ht + lax.dot_general(we1c_ref[...], sgtg, cdims)
    ht = ht + be1t_ref[...]
    ht = _gelu_exact(ht)
    logits = lax.dot_general(we2_ref[...], ht, cdims) + be2_ref[...]
    out_ref[esl] = jax.nn.sigmoid(logits)[0]


_edge_mlp = pl.pallas_call(
    _edge_mlp_body,
    grid=(EDGE_GRID,),
    in_specs=[
        pl.BlockSpec((D_EDGE, EDGE_BLOCK), lambda i: (0, i)),
        pl.BlockSpec((H, EDGE_BLOCK), lambda i: (0, i)),
        pl.BlockSpec((E,), lambda i: (0,)),
        pl.BlockSpec((E,), lambda i: (0,)),
        pl.BlockSpec((D_EDGE, D_EDGE), lambda i: (0, 0)),
        pl.BlockSpec((H, D_EDGE), lambda i: (0, 0)),
        pl.BlockSpec((2, D_EDGE), lambda i: (0, 0)),
        pl.BlockSpec((D_EDGE, 1), lambda i: (0, 0)),
        pl.BlockSpec((D_EDGE, 1), lambda i: (0, 0)),
        pl.BlockSpec((1, 1), lambda i: (0, 0)),
    ],
    out_specs=pl.BlockSpec((E,), lambda i: (0,)),
    out_shape=jax.ShapeDtypeStruct((E,), jnp.float32),
    compiler_params=pltpu.CompilerParams(
        dimension_semantics=("parallel",)),
)


def kernel(node_features, edge_features, edge_index, node_attn_weights,
           edge_attn_weights, Wn1, bn1, Wn2, bn2, We1, be1, We2, be2):
    attn_t = node_attn_weights.T

    zeros = jnp.zeros((H, N), jnp.float32)
    p0, p1 = _sc_scatter(edge_index, attn_t, zeros)

    node_gates = _node_mlp(
        p0, p1, node_features,
        Wn1[:D_NODE], Wn1[D_NODE:].T,
        bn1.reshape(-1, 1), Wn2, bn2.reshape(1, 1),
    )

    src_g, tgt_g = _sc_gather(edge_index, node_gates)

    edge_gates = _edge_mlp(
        edge_features.T, attn_t, src_g, tgt_g,
        We1[:D_EDGE], We1[D_EDGE:D_EDGE + H], We1[D_EDGE + H:],
        be1.reshape(-1, 1), We2, be2.reshape(1, 1),
    )

    return (node_gates, edge_gates)


# speedup vs baseline: 4.1784x; 1.0009x over previous
import functools

import jax
import jax.numpy as jnp
from jax import lax
from jax.experimental import pallas as pl
from jax.experimental.pallas import tpu as pltpu
from jax.experimental.pallas import tpu_sc as plsc

N = 10000
E = 320000
D_NODE = 128
D_EDGE = 16
H = 4

NUM_CORES = 2
NUM_SUBCORES = 16
NUM_TILES = NUM_CORES * NUM_SUBCORES
EDGES_PER_TILE = E // NUM_TILES
HALF_N = N // 2

_sc_mesh = plsc.VectorSubcoreMesh(core_axis_name="c", subcore_axis_name="s")
_sc_params = pltpu.CompilerParams(use_tc_tiling_on_sc=False,
                                  needs_layout_passes=False)

_INV_SQRT2 = 0.7071067811865476


def _gelu_exact(x):
    return x * 0.5 * (1.0 + lax.erf(x * _INV_SQRT2))


@functools.partial(
    pl.kernel,
    out_type=(
        jax.ShapeDtypeStruct((H, N), jnp.float32),
        jax.ShapeDtypeStruct((H, N), jnp.float32),
    ),
    mesh=_sc_mesh,
    compiler_params=_sc_params,
    scratch_types=[
        pltpu.VMEM((EDGES_PER_TILE,), jnp.int32),
        pltpu.VMEM((EDGES_PER_TILE,), jnp.float32),
        pltpu.VMEM_SHARED((H, N), jnp.float32),
    ],
)
def _sc_scatter(ei_hbm, attnt_hbm, zeros_hbm, out0_hbm, out1_hbm,
                idx_v, vals_v, acc_sh):
    c = lax.axis_index("c")
    s = lax.axis_index("s")
    wid = c * NUM_SUBCORES + s

    @pl.when(s < 2 * H)
    def _():
        h = s // 2
        sl = pl.ds((s % 2) * HALF_N, HALF_N)
        pltpu.sync_copy(zeros_hbm.at[h].at[sl], acc_sh.at[h].at[sl])

    base = wid * EDGES_PER_TILE
    pltpu.sync_copy(ei_hbm.at[1].at[pl.ds(base, EDGES_PER_TILE)], idx_v)

    plsc.subcore_barrier()

    for h in range(H):
        pltpu.sync_copy(attnt_hbm.at[h].at[pl.ds(base, EDGES_PER_TILE)],
                        vals_v)
        pltpu.sync_copy(vals_v, acc_sh.at[h].at[idx_v], add=True)

    plsc.subcore_barrier()

    @pl.when(s < 2 * H)
    def _():
        h = s // 2
        sl = pl.ds((s % 2) * HALF_N, HALF_N)

        @pl.when(c == 0)
        def _():
            pltpu.sync_copy(acc_sh.at[h].at[sl], out0_hbm.at[h].at[sl])

        @pl.when(c == 1)
        def _():
            pltpu.sync_copy(acc_sh.at[h].at[sl], out1_hbm.at[h].at[sl])


def _node_mlp_body(p0_ref, p1_ref, nf_ref, wn1a_ref, wn1bt_ref, be1t_ref,
                   wn2_ref, bn2_ref, out_ref):
    napt = p0_ref[...] + p1_ref[...]
    hm = jnp.max(napt, axis=1, keepdims=True) + 1e-10
    napnt = napt / hm
    h1t = lax.dot_general(wn1a_ref[...], nf_ref[...],
                          (((0,), (1,)), ((), ())))
    for hh in range(H):
        h1t = h1t + wn1bt_ref[:, hh:hh + 1] * napnt[hh:hh + 1, :]
    h1t = h1t + be1t_ref[...]
    h1t = _gelu_exact(h1t)
    logits = jnp.sum(h1t * wn2_ref[...], axis=0, keepdims=True) + bn2_ref[...]
    out_ref[...] = jax.nn.sigmoid(logits)[0]


_node_mlp = pl.pallas_call(
    _node_mlp_body,
    out_shape=jax.ShapeDtypeStruct((N,), jnp.float32),
)


@functools.partial(
    pl.kernel,
    out_type=(
        jax.ShapeDtypeStruct((E,), jnp.float32),
        jax.ShapeDtypeStruct((E,), jnp.float32),
    ),
    mesh=_sc_mesh,
    compiler_params=_sc_params,
    scratch_types=[
        pltpu.VMEM_SHARED((N,), jnp.float32),
        pltpu.VMEM((EDGES_PER_TILE,), jnp.int32),
        pltpu.VMEM((EDGES_PER_TILE,), jnp.int32),
        pltpu.VMEM((EDGES_PER_TILE,), jnp.float32),
        pltpu.VMEM((EDGES_PER_TILE,), jnp.float32),
    ],
)
def _sc_gather(ei_hbm, gates_hbm, outs_hbm, outt_hbm,
               gates_sh, sidx_v, tidx_v, souts_v, soutt_v):
    c = lax.axis_index("c")
    s = lax.axis_index("s")
    wid = c * NUM_SUBCORES + s
    base = wid * EDGES_PER_TILE

    @pl.when(s < 10)
    def _():
        sl = pl.ds(s * (N // 10), N // 10)
        pltpu.sync_copy(gates_hbm.at[sl], gates_sh.at[sl])

    pltpu.sync_copy(ei_hbm.at[0].at[pl.ds(base, EDGES_PER_TILE)], sidx_v)
    pltpu.sync_copy(ei_hbm.at[1].at[pl.ds(base, EDGES_PER_TILE)], tidx_v)

    plsc.subcore_barrier()

    pltpu.sync_copy(gates_sh.at[sidx_v], souts_v)
    pltpu.sync_copy(gates_sh.at[tidx_v], soutt_v)

    pltpu.sync_copy(souts_v, outs_hbm.at[pl.ds(base, EDGES_PER_TILE)])
    pltpu.sync_copy(soutt_v, outt_hbm.at[pl.ds(base, EDGES_PER_TILE)])


EDGE_BLOCK = 64000
EDGE_GRID = E // EDGE_BLOCK


def _edge_mlp_body(eft_ref, attnt_ref, sg_ref, tg_ref, we1a_ref, we1b_ref,
                   we1c_ref, be1t_ref, we2_ref, be2_ref, out_ref):
    i = pl.program_id(0)
    esl = pl.ds(i * EDGE_BLOCK, EDGE_BLOCK)
    cdims = (((0,), (0,)), ((), ()))
    ht = lax.dot_general(we1a_ref[...], eft_ref[...], cdims)
    ht = ht + lax.dot_general(we1b_ref[...], attnt_ref[...], cdims)
    sgtg = jnp.concatenate(
        [sg_ref[esl].reshape(1, EDGE_BLOCK), tg_ref[esl].reshape(1, EDGE_BLOCK)],
        axis=0)
    ht = ht + lax.dot_general(we1c_ref[...], sgtg, cdims)
    ht = ht + be1t_ref[...]
    ht = _gelu_exact(ht)
    logits = lax.dot_general(we2_ref[...], ht, cdims) + be2_ref[...]
    out_ref[esl] = jax.nn.sigmoid(logits)[0]


_edge_mlp = pl.pallas_call(
    _edge_mlp_body,
    grid=(EDGE_GRID,),
    in_specs=[
        pl.BlockSpec((D_EDGE, EDGE_BLOCK), lambda i: (0, i)),
        pl.BlockSpec((H, EDGE_BLOCK), lambda i: (0, i)),
        pl.BlockSpec((E,), lambda i: (0,)),
        pl.BlockSpec((E,), lambda i: (0,)),
        pl.BlockSpec((D_EDGE, D_EDGE), lambda i: (0, 0)),
        pl.BlockSpec((H, D_EDGE), lambda i: (0, 0)),
        pl.BlockSpec((2, D_EDGE), lambda i: (0, 0)),
        pl.BlockSpec((D_EDGE, 1), lambda i: (0, 0)),
        pl.BlockSpec((D_EDGE, 1), lambda i: (0, 0)),
        pl.BlockSpec((1, 1), lambda i: (0, 0)),
    ],
    out_specs=pl.BlockSpec((E,), lambda i: (0,)),
    out_shape=jax.ShapeDtypeStruct((E,), jnp.float32),
    compiler_params=pltpu.CompilerParams(
        dimension_semantics=("parallel",)),
)


def kernel(node_features, edge_features, edge_index, node_attn_weights,
           edge_attn_weights, Wn1, bn1, Wn2, bn2, We1, be1, We2, be2):
    attn_t = node_attn_weights.T

    zeros = jnp.zeros((H, N), jnp.float32)
    p0, p1 = _sc_scatter(edge_index, attn_t, zeros)

    node_gates = _node_mlp(
        p0, p1, node_features,
        Wn1[:D_NODE], Wn1[D_NODE:].T,
        bn1.reshape(-1, 1), Wn2, bn2.reshape(1, 1),
    )

    src_g, tgt_g = _sc_gather(edge_index, node_gates)

    edge_gates = _edge_mlp(
        edge_features.T, attn_t, src_g, tgt_g,
        We1[:D_EDGE], We1[D_EDGE:D_EDGE + H], We1[D_EDGE + H:],
        be1.reshape(-1, 1), We2, be2.reshape(1, 1),
    )

    return (node_gates, edge_gates)
